# baseline (speedup 1.0000x reference)
import functools

import jax
import jax.numpy as jnp
import numpy as np
from jax import lax
from jax.experimental import pallas as pl
from jax.experimental.pallas import tpu as pltpu
from jax.experimental.pallas import tpu_sc as plsc

_SCALE = 2
_NBUF = 4


def _route_indices(weight, out_channels, total):
    t2 = np.float32(np.float64(total) * np.float64(total))
    p = weight[:out_channels].astype(jnp.float32) * t2
    t = jnp.trunc(p)
    oc = np.float32(out_channels)
    r = jnp.fmod(t, oc)
    r = jnp.where(r < 0, r + oc, r).astype(jnp.int32)
    return r[:, 0], r[:, 1]


def _sc_body(x_hbm, wr_hbm, sr_hbm, xs_hbm, out_hbm,
             buf, wbuf, sbuf, vgbuf, sems, *, nb, b_n, oc, ots, rpw, nc):
    w = lax.axis_index("s") * nc + lax.axis_index("c")

    pltpu.sync_copy(wr_hbm.at[w], wbuf)
    pltpu.sync_copy(sr_hbm.at[w], sbuf)

    wchunk = wbuf[...]
    schunk = sbuf[...]
    lane = lax.broadcasted_iota(jnp.int32, (16,), 0)
    nfill = ots // 16

    def row_of(t):
        b, rr = divmod(t, rpw)
        return b * oc + w * rpw + rr

    def start_in(t):
        s = t % nb
        return pltpu.async_copy(x_hbm.at[row_of(t)], buf.at[s], sems.at[s])

    def start_out(t):
        s = t % nb
        return pltpu.async_copy(buf.at[s], out_hbm.at[row_of(t)],
                                sems.at[nb + s])

    ntask = b_n * rpw
    in_h = {}
    out_h = {}
    for t in range(min(nb, ntask)):
        in_h[t] = start_in(t)

    for b in range(b_n):
        fidx = jnp.full((16,), b * oc, jnp.int32) + schunk
        pltpu.async_copy(xs_hbm.at[fidx], vgbuf, sems.at[2 * nb]).wait()
        vchunk = vgbuf[...]
        for rr in range(rpw):
            t = b * rpw + rr
            s = t % nb
            in_h[t].wait()
            w_s = wchunk[rr]

            @pl.when(w_s != 0)
            def _fill():
                vfull = lax.gather(
                    vchunk, jnp.full((16, 1), rr, jnp.int32),
                    lax.GatherDimensionNumbers(
                        offset_dims=(), collapsed_slice_dims=(0,),
                        start_index_map=(0,)),
                    slice_sizes=(1,),
                    mode=lax.GatherScatterMode.PROMISE_IN_BOUNDS)

                bslot = buf.at[s]

                def fb(i, c):
                    bslot[pl.ds(i * 16, 16)] = vfull
                    return c

                lax.fori_loop(0, nfill, fb, 0)

            out_h[t] = start_out(t)
            nt = t + nb
            if nt < ntask:
                out_h[t].wait()
                in_h[nt] = start_in(nt)

    for t in range(max(0, ntask - nb), ntask):
        out_h[t].wait()


def kernel(x, weight):
    B, T, C = x.shape
    oc = C // _SCALE
    ots = T * _SCALE
    total = C * T - 1

    idx1, idx2 = _route_indices(weight, oc, total)
    ii = jnp.arange(oc, dtype=jnp.int32)
    last_i = jnp.full((oc,), -1, jnp.int32).at[idx1].max(ii)
    written = (last_i >= 0).astype(jnp.int32)
    srcs = idx2[jnp.clip(last_i, 0, oc - 1)]

    info = plsc.get_sparse_core_info()
    nc, ns = info.num_cores, info.num_subcores
    nw = nc * ns
    rpw = oc // nw

    wr_pad = jnp.zeros((nw, 16), jnp.int32).at[:, :rpw].set(
        written.reshape(nw, rpw))
    sr_pad = jnp.zeros((nw, 16), jnp.int32).at[:, :rpw].set(
        srcs.reshape(nw, rpw))
    xs = x[:, 0, :oc].reshape(-1)
    x2 = x.reshape(B * oc, ots)

    mesh = plsc.VectorSubcoreMesh(core_axis_name="c", subcore_axis_name="s")
    body = functools.partial(_sc_body, nb=_NBUF, b_n=B, oc=oc, ots=ots,
                             rpw=rpw, nc=nc)
    out2 = pl.kernel(
        body,
        mesh=mesh,
        out_type=jax.ShapeDtypeStruct((B * oc, ots), jnp.float32),
        scratch_types=[
            pltpu.VMEM((_NBUF, ots), jnp.float32),
            pltpu.VMEM((16,), jnp.int32),
            pltpu.VMEM((16,), jnp.int32),
            pltpu.VMEM((16,), jnp.float32),
            pltpu.SemaphoreType.DMA((2 * _NBUF + 1,)),
        ],
    )(x2, wr_pad, sr_pad, xs)
    return out2.reshape(B, oc, ots)


# speedup vs baseline: 1.0107x; 1.0107x over previous
import functools

import jax
import jax.numpy as jnp
import numpy as np
from jax import lax
from jax.experimental import pallas as pl
from jax.experimental.pallas import tpu as pltpu
from jax.experimental.pallas import tpu_sc as plsc

_SCALE = 2
_NBUF = 8
_LAG = 4


def _route_indices(weight, out_channels, total):
    t2 = np.float32(np.float64(total) * np.float64(total))
    p = weight[:out_channels].astype(jnp.float32) * t2
    t = jnp.trunc(p)
    oc = np.float32(out_channels)
    r = jnp.fmod(t, oc)
    r = jnp.where(r < 0, r + oc, r).astype(jnp.int32)
    return r[:, 0], r[:, 1]


def _sc_body(x_hbm, wr_hbm, sr_hbm, xs_hbm, out_hbm,
             buf, wbuf, sbuf, vgbuf, sems, *, nb, b_n, oc, ots, rpw, nc):
    w = lax.axis_index("s") * nc + lax.axis_index("c")

    pltpu.sync_copy(wr_hbm.at[w], wbuf)
    pltpu.sync_copy(sr_hbm.at[w], sbuf)

    wchunk = wbuf[...]
    schunk = sbuf[...]
    lane = lax.broadcasted_iota(jnp.int32, (16,), 0)
    nfill = ots // 16

    def row_of(t):
        b, rr = divmod(t, rpw)
        return b * oc + w * rpw + rr

    def start_in(t):
        s = t % nb
        return pltpu.async_copy(x_hbm.at[row_of(t)], buf.at[s], sems.at[s])

    def start_out(t):
        s = t % nb
        return pltpu.async_copy(buf.at[s], out_hbm.at[row_of(t)],
                                sems.at[nb + s])

    ntask = b_n * rpw
    lag = _LAG
    in_h = {}
    out_h = {}
    waited = set()
    for t in range(min(nb - lag, ntask)):
        in_h[t] = start_in(t)

    for b in range(b_n):
        fidx = jnp.full((16,), b * oc, jnp.int32) + schunk
        pltpu.async_copy(xs_hbm.at[fidx], vgbuf, sems.at[2 * nb]).wait()
        vchunk = vgbuf[...]
        for rr in range(rpw):
            t = b * rpw + rr
            s = t % nb
            in_h[t].wait()
            w_s = wchunk[rr]

            @pl.when(w_s != 0)
            def _fill():
                vfull = lax.gather(
                    vchunk, jnp.full((16, 1), rr, jnp.int32),
                    lax.GatherDimensionNumbers(
                        offset_dims=(), collapsed_slice_dims=(0,),
                        start_index_map=(0,)),
                    slice_sizes=(1,),
                    mode=lax.GatherScatterMode.PROMISE_IN_BOUNDS)

                bslot = buf.at[s]

                def fb(i, c):
                    bslot[pl.ds(i * 16, 16)] = vfull
                    return c

                lax.fori_loop(0, nfill, fb, 0)

            out_h[t] = start_out(t)
            nt = t + nb - lag
            if nt < ntask:
                if t >= lag:
                    out_h[t - lag].wait()
                    waited.add(t - lag)
                in_h[nt] = start_in(nt)

    for t in range(ntask):
        if t not in waited:
            out_h[t].wait()


def kernel(x, weight):
    B, T, C = x.shape
    oc = C // _SCALE
    ots = T * _SCALE
    total = C * T - 1

    idx1, idx2 = _route_indices(weight, oc, total)
    ii = jnp.arange(oc, dtype=jnp.int32)
    last_i = jnp.full((oc,), -1, jnp.int32).at[idx1].max(ii)
    written = (last_i >= 0).astype(jnp.int32)
    srcs = idx2[jnp.clip(last_i, 0, oc - 1)]

    info = plsc.get_sparse_core_info()
    nc, ns = info.num_cores, info.num_subcores
    nw = nc * ns
    rpw = oc // nw

    wr_pad = jnp.zeros((nw, 16), jnp.int32).at[:, :rpw].set(
        written.reshape(nw, rpw))
    sr_pad = jnp.zeros((nw, 16), jnp.int32).at[:, :rpw].set(
        srcs.reshape(nw, rpw))
    xs = x[:, 0, :oc].reshape(-1)
    x2 = x.reshape(B * oc, ots)

    mesh = plsc.VectorSubcoreMesh(core_axis_name="c", subcore_axis_name="s")
    body = functools.partial(_sc_body, nb=_NBUF, b_n=B, oc=oc, ots=ots,
                             rpw=rpw, nc=nc)
    out2 = pl.kernel(
        body,
        mesh=mesh,
        out_type=jax.ShapeDtypeStruct((B * oc, ots), jnp.float32),
        scratch_types=[
            pltpu.VMEM((_NBUF, ots), jnp.float32),
            pltpu.VMEM((16,), jnp.int32),
            pltpu.VMEM((16,), jnp.int32),
            pltpu.VMEM((16,), jnp.float32),
            pltpu.SemaphoreType.DMA((2 * _NBUF + 1,)),
        ],
    )(x2, wr_pad, sr_pad, xs)
    return out2.reshape(B, oc, ots)


# speedup vs baseline: 1.0132x; 1.0024x over previous
import functools

import jax
import jax.numpy as jnp
import numpy as np
from jax import lax
from jax.experimental import pallas as pl
from jax.experimental.pallas import tpu as pltpu
from jax.experimental.pallas import tpu_sc as plsc

_SCALE = 2
_NBUF = 3
_CHUNK_ROWS = 4


def _route_indices(weight, out_channels, total):
    t2 = np.float32(np.float64(total) * np.float64(total))
    p = weight[:out_channels].astype(jnp.float32) * t2
    t = jnp.trunc(p)
    oc = np.float32(out_channels)
    r = jnp.fmod(t, oc)
    r = jnp.where(r < 0, r + oc, r).astype(jnp.int32)
    return r[:, 0], r[:, 1]


def _sc_body(x_hbm, wr_hbm, sr_hbm, xs_hbm, out_hbm,
             buf, wbuf, sbuf, vgbuf, fbuf, sems, *, nb, b_n, oc, ots, rpw,
             nc, cr):
    w = lax.axis_index("s") * nc + lax.axis_index("c")

    pltpu.sync_copy(wr_hbm.at[w], wbuf)
    pltpu.sync_copy(sr_hbm.at[w], sbuf)

    wchunk = wbuf[...]
    schunk = sbuf[...]
    nfill = ots // 16
    nch = rpw // cr
    ntask = b_n * nch
    lag = 1

    def base_of(t):
        b, c = divmod(t, nch)
        return b * oc + w * rpw + c * cr

    def start_in(t):
        s = t % nb
        return pltpu.async_copy(x_hbm.at[pl.ds(base_of(t), cr)], buf.at[s],
                                sems.at[s])

    def start_out(t):
        s = t % nb
        return pltpu.async_copy(buf.at[s], out_hbm.at[pl.ds(base_of(t), cr)],
                                sems.at[nb + s])

    in_h = {}
    out_h = {}
    waited = set()
    for t in range(min(nb - lag, ntask)):
        in_h[t] = start_in(t)
    for t in range(ntask):
        in_h[t].wait()
        out_h[t] = start_out(t)
        nt = t + nb - lag
        if nt < ntask:
            if t >= lag:
                out_h[t - lag].wait()
                waited.add(t - lag)
            in_h[nt] = start_in(nt)
    for t in range(ntask):
        if t not in waited:
            out_h[t].wait()

    for b in range(b_n):
        fidx = jnp.full((16,), b * oc, jnp.int32) + schunk
        pltpu.async_copy(xs_hbm.at[fidx], vgbuf, sems.at[2 * nb]).wait()
        vchunk = vgbuf[...]
        for rr in range(rpw):
            w_s = wchunk[rr]

            @pl.when(w_s != 0)
            def _fill():
                vfull = lax.gather(
                    vchunk, jnp.full((16, 1), rr, jnp.int32),
                    lax.GatherDimensionNumbers(
                        offset_dims=(), collapsed_slice_dims=(0,),
                        start_index_map=(0,)),
                    slice_sizes=(1,),
                    mode=lax.GatherScatterMode.PROMISE_IN_BOUNDS)

                def fb(i, c):
                    fbuf[pl.ds(i * 16, 16)] = vfull
                    return c

                lax.fori_loop(0, nfill, fb, 0)
                pltpu.sync_copy(fbuf, out_hbm.at[b * oc + w * rpw + rr])


def kernel(x, weight):
    B, T, C = x.shape
    oc = C // _SCALE
    ots = T * _SCALE
    total = C * T - 1

    idx1, idx2 = _route_indices(weight, oc, total)
    ii = jnp.arange(oc, dtype=jnp.int32)
    last_i = jnp.full((oc,), -1, jnp.int32).at[idx1].max(ii)
    written = (last_i >= 0).astype(jnp.int32)
    srcs = idx2[jnp.clip(last_i, 0, oc - 1)]

    info = plsc.get_sparse_core_info()
    nc, ns = info.num_cores, info.num_subcores
    nw = nc * ns
    rpw = oc // nw

    wr_pad = jnp.zeros((nw, 16), jnp.int32).at[:, :rpw].set(
        written.reshape(nw, rpw))
    sr_pad = jnp.zeros((nw, 16), jnp.int32).at[:, :rpw].set(
        srcs.reshape(nw, rpw))
    xs = x[:, 0, :oc].reshape(-1)
    x2 = x.reshape(B * oc, ots)

    mesh = plsc.VectorSubcoreMesh(core_axis_name="c", subcore_axis_name="s")
    body = functools.partial(_sc_body, nb=_NBUF, b_n=B, oc=oc, ots=ots,
                             rpw=rpw, nc=nc, cr=_CHUNK_ROWS)
    out2 = pl.kernel(
        body,
        mesh=mesh,
        out_type=jax.ShapeDtypeStruct((B * oc, ots), jnp.float32),
        scratch_types=[
            pltpu.VMEM((_NBUF, _CHUNK_ROWS, ots), jnp.float32),
            pltpu.VMEM((16,), jnp.int32),
            pltpu.VMEM((16,), jnp.int32),
            pltpu.VMEM((16,), jnp.float32),
            pltpu.VMEM((ots,), jnp.float32),
            pltpu.SemaphoreType.DMA((2 * _NBUF + 1,)),
        ],
    )(x2, wr_pad, sr_pad, xs)
    return out2.reshape(B, oc, ots)


# speedup vs baseline: 1.0380x; 1.0245x over previous
import functools

import jax
import jax.numpy as jnp
import numpy as np
from jax import lax
from jax.experimental import pallas as pl
from jax.experimental.pallas import tpu as pltpu
from jax.experimental.pallas import tpu_sc as plsc

_SCALE = 2
_NBUF = 3
_CHUNK_ROWS = 4


def _route_indices(weight, out_channels, total):
    t2 = np.float32(np.float64(total) * np.float64(total))
    p = weight[:out_channels].astype(jnp.float32) * t2
    t = jnp.trunc(p)
    oc = np.float32(out_channels)
    r = jnp.fmod(t, oc)
    r = jnp.where(r < 0, r + oc, r).astype(jnp.int32)
    return r[:, 0], r[:, 1]


def _sc_body(x_hbm, wr_hbm, sr_hbm, xs_hbm, out_hbm,
             sbig, wbuf, sbuf, vgbuf, fbuf, sems, *, nb, b_n, oc, ots, rpw,
             nc, cr):
    sid = lax.axis_index("s")
    w = sid * nc + lax.axis_index("c")
    buf = sbig.at[sid]

    pltpu.sync_copy(wr_hbm.at[w], wbuf)
    pltpu.sync_copy(sr_hbm.at[w], sbuf)

    wchunk = wbuf[...]
    schunk = sbuf[...]
    nfill = ots // 16
    nch = rpw // cr
    ntask = b_n * nch
    lag = 1

    def base_of(t):
        b, c = divmod(t, nch)
        return b * oc + w * rpw + c * cr

    def start_in(t):
        s = t % nb
        return pltpu.async_copy(x_hbm.at[pl.ds(base_of(t), cr)], buf.at[s],
                                sems.at[s])

    def start_out(t):
        s = t % nb
        return pltpu.async_copy(buf.at[s], out_hbm.at[pl.ds(base_of(t), cr)],
                                sems.at[nb + s])

    in_h = {}
    out_h = {}
    waited = set()
    for t in range(min(nb - lag, ntask)):
        in_h[t] = start_in(t)
    for t in range(ntask):
        in_h[t].wait()
        out_h[t] = start_out(t)
        nt = t + nb - lag
        if nt < ntask:
            if t >= lag:
                out_h[t - lag].wait()
                waited.add(t - lag)
            in_h[nt] = start_in(nt)
    for t in range(ntask):
        if t not in waited:
            out_h[t].wait()

    for b in range(b_n):
        fidx = jnp.full((16,), b * oc, jnp.int32) + schunk
        pltpu.async_copy(xs_hbm.at[fidx], vgbuf, sems.at[2 * nb]).wait()
        vchunk = vgbuf[...]
        for rr in range(rpw):
            w_s = wchunk[rr]

            @pl.when(w_s != 0)
            def _fill():
                vfull = lax.gather(
                    vchunk, jnp.full((16, 1), rr, jnp.int32),
                    lax.GatherDimensionNumbers(
                        offset_dims=(), collapsed_slice_dims=(0,),
                        start_index_map=(0,)),
                    slice_sizes=(1,),
                    mode=lax.GatherScatterMode.PROMISE_IN_BOUNDS)

                def fb(i, c):
                    fbuf[pl.ds(i * 16, 16)] = vfull
                    return c

                lax.fori_loop(0, nfill, fb, 0)
                pltpu.sync_copy(fbuf, out_hbm.at[b * oc + w * rpw + rr])


def kernel(x, weight):
    B, T, C = x.shape
    oc = C // _SCALE
    ots = T * _SCALE
    total = C * T - 1

    idx1, idx2 = _route_indices(weight, oc, total)
    ii = jnp.arange(oc, dtype=jnp.int32)
    last_i = jnp.full((oc,), -1, jnp.int32).at[idx1].max(ii)
    written = (last_i >= 0).astype(jnp.int32)
    srcs = idx2[jnp.clip(last_i, 0, oc - 1)]

    info = plsc.get_sparse_core_info()
    nc, ns = info.num_cores, info.num_subcores
    nw = nc * ns
    rpw = oc // nw

    wr_pad = jnp.zeros((nw, 16), jnp.int32).at[:, :rpw].set(
        written.reshape(nw, rpw))
    sr_pad = jnp.zeros((nw, 16), jnp.int32).at[:, :rpw].set(
        srcs.reshape(nw, rpw))
    xs = x[:, 0, :oc].reshape(-1)
    x2 = x.reshape(B * oc, ots)

    mesh = plsc.VectorSubcoreMesh(core_axis_name="c", subcore_axis_name="s")
    body = functools.partial(_sc_body, nb=_NBUF, b_n=B, oc=oc, ots=ots,
                             rpw=rpw, nc=nc, cr=_CHUNK_ROWS)
    out2 = pl.kernel(
        body,
        mesh=mesh,
        out_type=jax.ShapeDtypeStruct((B * oc, ots), jnp.float32),
        scratch_types=[
            pltpu.VMEM_SHARED((16, _NBUF, _CHUNK_ROWS, ots), jnp.float32),
            pltpu.VMEM((16,), jnp.int32),
            pltpu.VMEM((16,), jnp.int32),
            pltpu.VMEM((16,), jnp.float32),
            pltpu.VMEM((ots,), jnp.float32),
            pltpu.SemaphoreType.DMA((2 * _NBUF + 1,)),
        ],
    )(x2, wr_pad, sr_pad, xs)
    return out2.reshape(B, oc, ots)
